# Initial kernel scaffold; baseline (speedup 1.0000x reference)
import jax
import jax.numpy as jnp
from jax.experimental import pallas as pl


def kernel(x_user, x_item, click_src, click_dst, dislike_src, dislike_dst, follow_src, follow_dst, W1, b1, W2, b2, Wp, bp):
    raise NotImplementedError("write your pallas kernel here")


# speedup vs baseline: 2.8532x; 2.8532x over previous
import functools

import jax
import jax.numpy as jnp
from jax import lax
from jax.experimental import pallas as pl
from jax.experimental.pallas import tpu as pltpu
from jax.experimental.pallas import tpu_sc as plsc

NU = 10000
NP = 10240
D = 128
CH = 128
EF, EC, ED = 96000, 160000, 32000
RPT = NP // 16
BLK = 1024
GRID = NP // BLK

_mesh = plsc.VectorSubcoreMesh(core_axis_name="c", subcore_axis_name="s")
_f32 = jnp.float32


def _chunks_for(worker, nworkers, n_chunks):
    nfull, rem = n_chunks // nworkers, n_chunks % nworkers
    return nfull + jnp.where(worker < rem, 1, 0)


@functools.partial(
    pl.kernel,
    out_type=[jax.ShapeDtypeStruct((NP, D), _f32)] * 6,
    mesh=_mesh,
    scratch_types=[
        pltpu.VMEM((CH,), jnp.int32),
        pltpu.VMEM((CH, D), _f32),
        pltpu.VMEM_SHARED((NP, D), _f32),
    ],
)
def _deg_kernel(fs, fd, cs, cd, dsk, dd, ones_hbm, zrows_hbm,
                dfs, dfd, dcs, dcd, dds, ddd,
                idx_v, ones_v, hist):
    c = lax.axis_index("c")
    s = lax.axis_index("s")
    row0 = pl.multiple_of(s * RPT, 8)
    pltpu.sync_copy(ones_hbm, ones_v)
    plan = [
        (0, fs, EF, dfs), (0, cs, EC, dcs), (0, dsk, ED, dds),
        (1, fd, EF, dfd), (1, cd, EC, dcd), (1, dd, ED, ddd),
    ]
    for core, arr, E, out in plan:
        def run(arr=arr, E=E, out=out):
            pltpu.sync_copy(zrows_hbm, hist.at[pl.ds(row0, RPT)])
            plsc.subcore_barrier()
            n_s = _chunks_for(s, 16, E // CH)

            def body(j, carry):
                base = pl.multiple_of((s + 16 * j) * CH, CH)
                pltpu.sync_copy(arr.at[pl.ds(base, CH)], idx_v)
                pltpu.sync_copy(ones_v, hist.at[idx_v], add=True)
                return carry

            lax.fori_loop(0, n_s, body, 0)
            plsc.subcore_barrier()
            pltpu.sync_copy(hist.at[pl.ds(row0, RPT)], out.at[pl.ds(row0, RPT)])
        pl.when(c == core)(run)


@functools.partial(
    pl.kernel,
    out_type=[jax.ShapeDtypeStruct((NP, D), _f32)] * 6,
    mesh=_mesh,
    scratch_types=[
        pltpu.VMEM((CH,), jnp.int32),
        pltpu.VMEM((CH,), jnp.int32),
        pltpu.VMEM((CH, D), _f32),
        pltpu.VMEM_SHARED((NP, D), _f32),
        pltpu.SemaphoreType.DMA,
    ],
)
def _agg_kernel(t0, t1, t2, t3, t4, t5, fs, fd, cs, cd, dsk, dd, zrows,
                y0, y1, y2, y3, y4, y5,
                sidx, didx, rows, acc, gsem):
    c = lax.axis_index("c")
    s = lax.axis_index("s")
    row0 = pl.multiple_of(s * RPT, 8)
    plan = [
        (0, t0, fs, fd, EF, y0),
        (0, t2, cs, cd, EC, y2),
        (0, t5, dd, dsk, ED, y5),
        (1, t1, fd, fs, EF, y1),
        (1, t3, cd, cs, EC, y3),
        (1, t4, dsk, dd, ED, y4),
    ]
    for core, tbl, src, dst, E, yout in plan:
        def run(tbl=tbl, src=src, dst=dst, E=E, yout=yout):
            pltpu.sync_copy(zrows, acc.at[pl.ds(row0, RPT)])
            plsc.subcore_barrier()
            n_s = _chunks_for(s, 16, E // CH)

            def body(j, carry):
                base = pl.multiple_of((s + 16 * j) * CH, CH)
                pltpu.sync_copy(src.at[pl.ds(base, CH)], sidx)
                pltpu.sync_copy(dst.at[pl.ds(base, CH)], didx)
                pltpu.async_copy(tbl.at[sidx], rows, gsem).wait()
                pltpu.sync_copy(rows, acc.at[didx], add=True)
                return carry

            lax.fori_loop(0, n_s, body, 0)
            plsc.subcore_barrier()
            pltpu.sync_copy(acc.at[pl.ds(row0, RPT)], yout.at[pl.ds(row0, RPT)])
        pl.when(c == core)(run)


_NOUT = (EC + ED) // 8


@functools.partial(
    pl.kernel,
    out_type=jax.ShapeDtypeStruct((_NOUT, D), _f32),
    mesh=_mesh,
    scratch_types=[
        pltpu.VMEM((CH,), jnp.int32),
        pltpu.VMEM((CH,), jnp.int32),
        pltpu.VMEM((CH, D), _f32),
        pltpu.VMEM((CH, D), _f32),
        pltpu.VMEM((CH // 8, D), _f32),
        pltpu.SemaphoreType.DMA,
    ],
)
def _pred_kernel(pu, pi, cs, cd, dsk, dd, out,
                 sidx, didx, arows, brows, crows, sem):
    c = lax.axis_index("c")
    s = lax.axis_index("s")
    w = s * 2 + c
    for arr_s, arr_d, E, obase in [(cs, cd, EC, 0), (dsk, dd, ED, EC)]:
        def seg(arr_s=arr_s, arr_d=arr_d, E=E, obase=obase):
            n_w = _chunks_for(w, 32, E // CH)

            def body(j, carry):
                base = pl.multiple_of((w + 32 * j) * CH, CH)
                pltpu.sync_copy(arr_s.at[pl.ds(base, CH)], sidx)
                pltpu.sync_copy(arr_d.at[pl.ds(base, CH)], didx)
                ca = pltpu.async_copy(pu.at[sidx], arows, sem)
                cb = pltpu.async_copy(pi.at[didx], brows, sem)
                ca.wait()
                cb.wait()
                for k in range(CH):
                    crows[k // 8, pl.ds((k % 8) * 16, 16)] = (
                        arows[k, pl.ds(0, 16)] + brows[k, pl.ds(0, 16)])
                orow = pl.multiple_of((obase + base) // 8, CH // 8)
                pltpu.sync_copy(crows, out.at[pl.ds(orow, CH // 8)])
                return carry

            lax.fori_loop(0, n_w, body, 0)
        seg()


def _tc_pre_body(xu, xi, dfs, dfd, dcs, dcd, dds, ddd,
                 t0, t1, t2, t3, t4, t5, sfs, sfd, scs, scd, sds, sdd):
    u = xu[...]
    it = xi[...]
    sv = []
    for dref, sref in [(dfs, sfs), (dfd, sfd), (dcs, scs), (dcd, scd),
                       (dds, sds), (ddd, sdd)]:
        v = lax.rsqrt(jnp.maximum(dref[...], 1.0))
        sref[...] = v
        sv.append(v)
    t0[...] = u * sv[0]
    t1[...] = u * sv[1]
    t2[...] = u * sv[2]
    t4[...] = u * sv[4]
    t3[...] = it * sv[3]
    t5[...] = it * sv[5]


def _layer_mats(y, sv, wu, wi, bu, bi):
    hu = jnp.concatenate(
        [y[0] * sv[1], y[1] * sv[0], y[3] * sv[2], y[5] * sv[4]], axis=1)
    hu = jnp.dot(hu, wu, preferred_element_type=_f32) + bu
    hi = jnp.concatenate([y[2] * sv[3], y[4] * sv[5]], axis=1)
    hi = jnp.dot(hi, wi, preferred_element_type=_f32) + bi
    return hu, hi


def _tc_mid_body(y0, y1, y2, y3, y4, y5, sfs, sfd, scs, scd, sds, sdd,
                 wu, wi, bu, bi,
                 o0, o1, o2, o3, o4, o5):
    sv = [sfs[...], sfd[...], scs[...], scd[...], sds[...], sdd[...]]
    hu, hi = _layer_mats([y0[...], y1[...], y2[...], y3[...], y4[...], y5[...]],
                         sv, wu[...], wi[...], bu[...], bi[...])
    hu = jnp.maximum(hu, 0.0)
    hi = jnp.maximum(hi, 0.0)
    o0[...] = hu * sv[0]
    o1[...] = hu * sv[1]
    o2[...] = hu * sv[2]
    o4[...] = hu * sv[4]
    o3[...] = hi * sv[3]
    o5[...] = hi * sv[5]


def _tc_post_body(y0, y1, y2, y3, y4, y5, sfs, sfd, scs, scd, sds, sdd,
                  wu, wi, bu, bi, wpu, wpi, bpp,
                  pu, pi):
    sv = [sfs[...], sfd[...], scs[...], scd[...], sds[...], sdd[...]]
    hu, hi = _layer_mats([y0[...], y1[...], y2[...], y3[...], y4[...], y5[...]],
                         sv, wu[...], wi[...], bu[...], bi[...])
    pu[...] = jnp.dot(hu, wpu[...], preferred_element_type=_f32) + bpp[...]
    pi[...] = jnp.dot(hi, wpi[...], preferred_element_type=_f32)


def _blk(shape):
    return pl.BlockSpec(shape, lambda i: (0,) * len(shape))


_rows_D = pl.BlockSpec((BLK, D), lambda i: (i, 0))
_rows_1 = pl.BlockSpec((BLK, 1), lambda i: (i, 0))
_rows_16 = pl.BlockSpec((BLK, 16), lambda i: (i, 0))

_tc_pre = pl.pallas_call(
    _tc_pre_body,
    grid=(GRID,),
    in_specs=[_rows_D] * 2 + [_rows_1] * 6,
    out_specs=[_rows_D] * 6 + [_rows_1] * 6,
    out_shape=[jax.ShapeDtypeStruct((NP, D), _f32)] * 6
              + [jax.ShapeDtypeStruct((NP, 1), _f32)] * 6,
)

_tc_mid = pl.pallas_call(
    _tc_mid_body,
    grid=(GRID,),
    in_specs=[_rows_D] * 6 + [_rows_1] * 6
             + [_blk((4 * D, D)), _blk((2 * D, D)), _blk((1, D)), _blk((1, D))],
    out_specs=[_rows_D] * 6,
    out_shape=[jax.ShapeDtypeStruct((NP, D), _f32)] * 6,
)

_tc_post = pl.pallas_call(
    _tc_post_body,
    grid=(GRID,),
    in_specs=[_rows_D] * 6 + [_rows_1] * 6
             + [_blk((4 * D, D)), _blk((2 * D, D)), _blk((1, D)), _blk((1, D)),
                _blk((D, D)), _blk((D, D)), _blk((1, D))],
    out_specs=[_rows_D] * 2,
    out_shape=[jax.ShapeDtypeStruct((NP, D), _f32)] * 2,
)


def kernel(x_user, x_item, click_src, click_dst, dislike_src, dislike_dst,
           follow_src, follow_dst, W1, b1, W2, b2, Wp, bp):
    xu = jnp.pad(x_user.astype(_f32), ((0, NP - NU), (0, 0)))
    xi = jnp.pad(x_item.astype(_f32), ((0, NP - NU), (0, 0)))
    ones128 = jnp.ones((CH, D), _f32)
    zrows = jnp.zeros((RPT, D), _f32)
    edges = (click_src, click_dst, dislike_src, dislike_dst,
             follow_src, follow_dst)
    cs, cd, dsk, dd, fs, fd = (e.astype(jnp.int32) for e in edges)

    degs = _deg_kernel(fs, fd, cs, cd, dsk, dd, ones128, zrows)
    dcols = [dg[:, :1] for dg in degs]

    pre = _tc_pre(xu, xi, *dcols)
    tabs1, svecs = pre[:6], pre[6:]

    ys1 = _agg_kernel(*tabs1, fs, fd, cs, cd, dsk, dd, zrows)

    w1u = jnp.concatenate([W1[0], W1[1], W1[3], W1[5]], axis=0)
    w1i = jnp.concatenate([W1[2], W1[4]], axis=0)
    b1u = (b1[0] + b1[1] + b1[3] + b1[5]).reshape(1, D)
    b1i = (b1[2] + b1[4]).reshape(1, D)
    tabs2 = _tc_mid(*ys1, *svecs, w1u, w1i, b1u, b1i)

    ys2 = _agg_kernel(*tabs2, fs, fd, cs, cd, dsk, dd, zrows)

    w2u = jnp.concatenate([W2[0], W2[1], W2[3], W2[5]], axis=0)
    w2i = jnp.concatenate([W2[2], W2[4]], axis=0)
    b2u = (b2[0] + b2[1] + b2[3] + b2[5]).reshape(1, D)
    b2i = (b2[2] + b2[4]).reshape(1, D)
    wpu = jnp.pad(Wp[:D], ((0, 0), (0, D - 6)))
    wpi = jnp.pad(Wp[D:], ((0, 0), (0, D - 6)))
    bpp = jnp.pad(bp, (0, D - 6)).reshape(1, D)
    pu, pi = _tc_post(*ys2, *svecs, w2u, w2i, b2u, b2i, wpu, wpi, bpp)

    packed = _pred_kernel(pu, pi, cs, cd, dsk, dd)
    return packed.reshape(EC + ED, 16)[:, :6]


# speedup vs baseline: 3.8779x; 1.3591x over previous
import functools

import jax
import jax.numpy as jnp
from jax import lax
from jax.experimental import pallas as pl
from jax.experimental.pallas import tpu as pltpu
from jax.experimental.pallas import tpu_sc as plsc

NU = 10000
NP = 10240
D = 128
CH = 128
CHA = 64
EF, EC, ED = 96000, 160000, 32000
RPT = NP // 16
BLK = 1024
GRID = NP // BLK

_mesh = plsc.VectorSubcoreMesh(core_axis_name="c", subcore_axis_name="s")
_f32 = jnp.float32


def _chunks_for(worker, nworkers, n_chunks):
    nfull, rem = n_chunks // nworkers, n_chunks % nworkers
    return nfull + jnp.where(worker < rem, 1, 0)


def _max_chunks(nworkers, n_chunks):
    return n_chunks // nworkers + (1 if n_chunks % nworkers else 0)


_Q = 4


@functools.partial(
    pl.kernel,
    out_type=[jax.ShapeDtypeStruct((NP, D), _f32)] * 6,
    mesh=_mesh,
    scratch_types=[
        pltpu.VMEM((_Q, CH), jnp.int32),
        pltpu.VMEM((CH, D), _f32),
        pltpu.VMEM_SHARED((NP, D), _f32),
    ] + [pltpu.SemaphoreType.DMA] * _Q,
)
def _deg_kernel(fs, fd, cs, cd, dsk, dd, ones_hbm, zrows_hbm,
                dfs, dfd, dcs, dcd, dds, ddd,
                idx_q, ones_v, hist, *sems):
    c = lax.axis_index("c")
    s = lax.axis_index("s")
    row0 = pl.multiple_of(s * RPT, 8)
    pltpu.sync_copy(ones_hbm, ones_v)
    plan = [
        (0, fs, EF, dfs), (0, cs, EC, dcs), (0, dsk, ED, dds),
        (1, fd, EF, dfd), (1, cd, EC, dcd), (1, dd, ED, ddd),
    ]
    for core, arr, E, out in plan:
        def run(arr=arr, E=E, out=out):
            pltpu.sync_copy(zrows_hbm, hist.at[pl.ds(row0, RPT)])
            plsc.subcore_barrier()
            n_s = _chunks_for(s, 16, E // CH)
            nq = -(-_max_chunks(16, E // CH) // _Q)

            def body(i, carry):
                conds, bases = [], []
                for k in range(_Q):
                    j = i * _Q + k
                    conds.append(j < n_s)
                    bases.append(pl.multiple_of((s + 16 * j) * CH, CH))
                for k in range(_Q):
                    def enq(k=k):
                        pltpu.async_copy(
                            arr.at[pl.ds(bases[k], CH)], idx_q.at[k], sems[k])
                    pl.when(conds[k])(enq)
                for k in range(_Q):
                    def scat(k=k):
                        pltpu.make_async_copy(
                            arr.at[pl.ds(bases[k], CH)], idx_q.at[k],
                            sems[k]).wait()
                        pltpu.async_copy(
                            ones_v, hist.at[idx_q.at[k]], sems[k], add=True)
                    pl.when(conds[k])(scat)
                for k in range(_Q):
                    def drain(k=k):
                        pltpu.make_async_copy(
                            ones_v, hist.at[idx_q.at[k]], sems[k]).wait()
                    pl.when(conds[k])(drain)
                return carry

            lax.fori_loop(0, nq, body, 0)
            plsc.subcore_barrier()
            pltpu.sync_copy(hist.at[pl.ds(row0, RPT)], out.at[pl.ds(row0, RPT)])
        pl.when(c == core)(run)


@functools.partial(
    pl.kernel,
    out_type=[jax.ShapeDtypeStruct((NP, D), _f32)] * 6,
    mesh=_mesh,
    scratch_types=[
        pltpu.VMEM((_Q, CHA), jnp.int32),
        pltpu.VMEM((_Q, CHA), jnp.int32),
    ] + [pltpu.VMEM((CHA, D), _f32)] * _Q + [
        pltpu.VMEM_SHARED((NP, D), _f32),
    ] + [pltpu.SemaphoreType.DMA] * _Q,
)
def _agg_kernel(t0, t1, t2, t3, t4, t5, fs, fd, cs, cd, dsk, dd, zrows,
                y0, y1, y2, y3, y4, y5,
                sidx_q, didx_q, *rest):
    rows_l = rest[:_Q]
    acc = rest[_Q]
    sems = rest[_Q + 1:]
    c = lax.axis_index("c")
    s = lax.axis_index("s")
    row0 = pl.multiple_of(s * RPT, 8)
    plan = [
        (0, t0, fs, fd, EF, y0),
        (0, t2, cs, cd, EC, y2),
        (0, t5, dd, dsk, ED, y5),
        (1, t1, fd, fs, EF, y1),
        (1, t3, cd, cs, EC, y3),
        (1, t4, dsk, dd, ED, y4),
    ]
    for core, tbl, src, dst, E, yout in plan:
        def run(tbl=tbl, src=src, dst=dst, E=E, yout=yout):
            pltpu.sync_copy(zrows, acc.at[pl.ds(row0, RPT)])
            plsc.subcore_barrier()
            n_s = _chunks_for(s, 16, E // CHA)
            nq = -(-_max_chunks(16, E // CHA) // _Q)

            def body(i, carry):
                conds, bases = [], []
                for k in range(_Q):
                    j = i * _Q + k
                    conds.append(j < n_s)
                    bases.append(pl.multiple_of((s + 16 * j) * CHA, CHA))
                for k in range(_Q):
                    def enq(k=k):
                        pltpu.async_copy(
                            src.at[pl.ds(bases[k], CHA)], sidx_q.at[k], sems[k])
                        pltpu.async_copy(
                            dst.at[pl.ds(bases[k], CHA)], didx_q.at[k], sems[k])
                    pl.when(conds[k])(enq)
                for k in range(_Q):
                    def gath(k=k):
                        pltpu.make_async_copy(
                            src.at[pl.ds(bases[k], CHA)], sidx_q.at[k],
                            sems[k]).wait()
                        pltpu.make_async_copy(
                            dst.at[pl.ds(bases[k], CHA)], didx_q.at[k],
                            sems[k]).wait()
                        pltpu.async_copy(
                            tbl.at[sidx_q.at[k]], rows_l[k], sems[k])
                    pl.when(conds[k])(gath)
                for k in range(_Q):
                    def scat(k=k):
                        pltpu.make_async_copy(
                            tbl.at[sidx_q.at[k]], rows_l[k], sems[k]).wait()
                        pltpu.async_copy(
                            rows_l[k], acc.at[didx_q.at[k]], sems[k],
                            add=True)
                    pl.when(conds[k])(scat)
                for k in range(_Q):
                    def drain(k=k):
                        pltpu.make_async_copy(
                            rows_l[k], acc.at[didx_q.at[k]], sems[k]).wait()
                    pl.when(conds[k])(drain)
                return carry

            lax.fori_loop(0, nq, body, 0)
            plsc.subcore_barrier()
            pltpu.sync_copy(acc.at[pl.ds(row0, RPT)], yout.at[pl.ds(row0, RPT)])
        pl.when(c == core)(run)


_NOUT = (EC + ED) // 8


_QP = 3


@functools.partial(
    pl.kernel,
    out_type=jax.ShapeDtypeStruct((_NOUT, D), _f32),
    mesh=_mesh,
    scratch_types=[
        pltpu.VMEM((_QP, CH), jnp.int32),
        pltpu.VMEM((_QP, CH), jnp.int32),
    ] + [pltpu.VMEM((CH, D), _f32)] * (2 * _QP)
      + [pltpu.VMEM((CH // 8, D), _f32)] * _QP
      + [pltpu.SemaphoreType.DMA] * _QP,
)
def _pred_kernel(pu, pi, cs, cd, dsk, dd, out,
                 sidx_q, didx_q, *rest):
    arows_l = rest[:_QP]
    brows_l = rest[_QP:2 * _QP]
    crows_l = rest[2 * _QP:3 * _QP]
    sems = rest[3 * _QP:]
    c = lax.axis_index("c")
    s = lax.axis_index("s")
    w = s * 2 + c
    for arr_s, arr_d, E, obase in [(cs, cd, EC, 0), (dsk, dd, ED, EC)]:
        def seg(arr_s=arr_s, arr_d=arr_d, E=E, obase=obase):
            n_w = _chunks_for(w, 32, E // CH)
            nq = -(-_max_chunks(32, E // CH) // _QP)

            def body(i, carry):
                conds, bases = [], []
                for k in range(_QP):
                    j = i * _QP + k
                    conds.append(j < n_w)
                    bases.append(pl.multiple_of((w + 32 * j) * CH, CH))
                for k in range(_QP):
                    def enq(k=k):
                        pltpu.async_copy(
                            arr_s.at[pl.ds(bases[k], CH)], sidx_q.at[k],
                            sems[k])
                        pltpu.async_copy(
                            arr_d.at[pl.ds(bases[k], CH)], didx_q.at[k],
                            sems[k])
                    pl.when(conds[k])(enq)
                for k in range(_QP):
                    def gath(k=k):
                        pltpu.make_async_copy(
                            arr_s.at[pl.ds(bases[k], CH)], sidx_q.at[k],
                            sems[k]).wait()
                        pltpu.make_async_copy(
                            arr_d.at[pl.ds(bases[k], CH)], didx_q.at[k],
                            sems[k]).wait()
                        pltpu.async_copy(
                            pu.at[sidx_q.at[k]], arows_l[k], sems[k])
                        pltpu.async_copy(
                            pi.at[didx_q.at[k]], brows_l[k], sems[k])
                    pl.when(conds[k])(gath)
                for k in range(_QP):
                    def pack(k=k):
                        pltpu.make_async_copy(
                            pu.at[sidx_q.at[k]], arows_l[k], sems[k]).wait()
                        pltpu.make_async_copy(
                            pi.at[didx_q.at[k]], brows_l[k], sems[k]).wait()
                        for r in range(CH):
                            crows_l[k][r // 8, pl.ds((r % 8) * 16, 16)] = (
                                arows_l[k][r, pl.ds(0, 16)]
                                + brows_l[k][r, pl.ds(0, 16)])
                        orow = pl.multiple_of((obase + bases[k]) // 8, CH // 8)
                        pltpu.async_copy(
                            crows_l[k], out.at[pl.ds(orow, CH // 8)],
                            sems[k])
                    pl.when(conds[k])(pack)
                for k in range(_QP):
                    def drain(k=k):
                        orow = pl.multiple_of((obase + bases[k]) // 8, CH // 8)
                        pltpu.make_async_copy(
                            crows_l[k], out.at[pl.ds(orow, CH // 8)],
                            sems[k]).wait()
                    pl.when(conds[k])(drain)
                return carry

            lax.fori_loop(0, nq, body, 0)
        seg()


def _tc_pre_body(xu, xi, dfs, dfd, dcs, dcd, dds, ddd,
                 t0, t1, t2, t3, t4, t5, sfs, sfd, scs, scd, sds, sdd):
    u = xu[...]
    it = xi[...]
    sv = []
    for dref, sref in [(dfs, sfs), (dfd, sfd), (dcs, scs), (dcd, scd),
                       (dds, sds), (ddd, sdd)]:
        v = lax.rsqrt(jnp.maximum(dref[...], 1.0))
        sref[...] = v
        sv.append(v)
    t0[...] = u * sv[0]
    t1[...] = u * sv[1]
    t2[...] = u * sv[2]
    t4[...] = u * sv[4]
    t3[...] = it * sv[3]
    t5[...] = it * sv[5]


def _layer_mats(y, sv, wu, wi, bu, bi):
    hu = jnp.concatenate(
        [y[0] * sv[1], y[1] * sv[0], y[3] * sv[2], y[5] * sv[4]], axis=1)
    hu = jnp.dot(hu, wu, preferred_element_type=_f32) + bu
    hi = jnp.concatenate([y[2] * sv[3], y[4] * sv[5]], axis=1)
    hi = jnp.dot(hi, wi, preferred_element_type=_f32) + bi
    return hu, hi


def _tc_mid_body(y0, y1, y2, y3, y4, y5, sfs, sfd, scs, scd, sds, sdd,
                 wu, wi, bu, bi,
                 o0, o1, o2, o3, o4, o5):
    sv = [sfs[...], sfd[...], scs[...], scd[...], sds[...], sdd[...]]
    hu, hi = _layer_mats([y0[...], y1[...], y2[...], y3[...], y4[...], y5[...]],
                         sv, wu[...], wi[...], bu[...], bi[...])
    hu = jnp.maximum(hu, 0.0)
    hi = jnp.maximum(hi, 0.0)
    o0[...] = hu * sv[0]
    o1[...] = hu * sv[1]
    o2[...] = hu * sv[2]
    o4[...] = hu * sv[4]
    o3[...] = hi * sv[3]
    o5[...] = hi * sv[5]


def _tc_post_body(y0, y1, y2, y3, y4, y5, sfs, sfd, scs, scd, sds, sdd,
                  wu, wi, bu, bi, wpu, wpi, bpp,
                  pu, pi):
    sv = [sfs[...], sfd[...], scs[...], scd[...], sds[...], sdd[...]]
    hu, hi = _layer_mats([y0[...], y1[...], y2[...], y3[...], y4[...], y5[...]],
                         sv, wu[...], wi[...], bu[...], bi[...])
    pu[...] = jnp.dot(hu, wpu[...], preferred_element_type=_f32) + bpp[...]
    pi[...] = jnp.dot(hi, wpi[...], preferred_element_type=_f32)


def _blk(shape):
    return pl.BlockSpec(shape, lambda i: (0,) * len(shape))


_rows_D = pl.BlockSpec((BLK, D), lambda i: (i, 0))
_rows_1 = pl.BlockSpec((BLK, 1), lambda i: (i, 0))
_rows_16 = pl.BlockSpec((BLK, 16), lambda i: (i, 0))

_tc_pre = pl.pallas_call(
    _tc_pre_body,
    grid=(GRID,),
    in_specs=[_rows_D] * 2 + [_rows_1] * 6,
    out_specs=[_rows_D] * 6 + [_rows_1] * 6,
    out_shape=[jax.ShapeDtypeStruct((NP, D), _f32)] * 6
              + [jax.ShapeDtypeStruct((NP, 1), _f32)] * 6,
)

_tc_mid = pl.pallas_call(
    _tc_mid_body,
    grid=(GRID,),
    in_specs=[_rows_D] * 6 + [_rows_1] * 6
             + [_blk((4 * D, D)), _blk((2 * D, D)), _blk((1, D)), _blk((1, D))],
    out_specs=[_rows_D] * 6,
    out_shape=[jax.ShapeDtypeStruct((NP, D), _f32)] * 6,
)

_tc_post = pl.pallas_call(
    _tc_post_body,
    grid=(GRID,),
    in_specs=[_rows_D] * 6 + [_rows_1] * 6
             + [_blk((4 * D, D)), _blk((2 * D, D)), _blk((1, D)), _blk((1, D)),
                _blk((D, D)), _blk((D, D)), _blk((1, D))],
    out_specs=[_rows_D] * 2,
    out_shape=[jax.ShapeDtypeStruct((NP, D), _f32)] * 2,
)


def kernel(x_user, x_item, click_src, click_dst, dislike_src, dislike_dst,
           follow_src, follow_dst, W1, b1, W2, b2, Wp, bp):
    xu = jnp.pad(x_user.astype(_f32), ((0, NP - NU), (0, 0)))
    xi = jnp.pad(x_item.astype(_f32), ((0, NP - NU), (0, 0)))
    ones128 = jnp.ones((CH, D), _f32)
    zrows = jnp.zeros((RPT, D), _f32)
    edges = (click_src, click_dst, dislike_src, dislike_dst,
             follow_src, follow_dst)
    cs, cd, dsk, dd, fs, fd = (e.astype(jnp.int32) for e in edges)

    degs = _deg_kernel(fs, fd, cs, cd, dsk, dd, ones128, zrows)
    dcols = [dg[:, :1] for dg in degs]

    pre = _tc_pre(xu, xi, *dcols)
    tabs1, svecs = pre[:6], pre[6:]

    ys1 = _agg_kernel(*tabs1, fs, fd, cs, cd, dsk, dd, zrows)

    w1u = jnp.concatenate([W1[0], W1[1], W1[3], W1[5]], axis=0)
    w1i = jnp.concatenate([W1[2], W1[4]], axis=0)
    b1u = (b1[0] + b1[1] + b1[3] + b1[5]).reshape(1, D)
    b1i = (b1[2] + b1[4]).reshape(1, D)
    tabs2 = _tc_mid(*ys1, *svecs, w1u, w1i, b1u, b1i)

    ys2 = _agg_kernel(*tabs2, fs, fd, cs, cd, dsk, dd, zrows)

    w2u = jnp.concatenate([W2[0], W2[1], W2[3], W2[5]], axis=0)
    w2i = jnp.concatenate([W2[2], W2[4]], axis=0)
    b2u = (b2[0] + b2[1] + b2[3] + b2[5]).reshape(1, D)
    b2i = (b2[2] + b2[4]).reshape(1, D)
    wpu = jnp.pad(Wp[:D], ((0, 0), (0, D - 6)))
    wpi = jnp.pad(Wp[D:], ((0, 0), (0, D - 6)))
    bpp = jnp.pad(bp, (0, D - 6)).reshape(1, D)
    pu, pi = _tc_post(*ys2, *svecs, w2u, w2i, b2u, b2i, wpu, wpi, bpp)

    packed = _pred_kernel(pu, pi, cs, cd, dsk, dd)
    return packed.reshape(EC + ED, 16)[:, :6]


# speedup vs baseline: 3.9099x; 1.0083x over previous
import functools

import jax
import jax.numpy as jnp
from jax import lax
from jax.experimental import pallas as pl
from jax.experimental.pallas import tpu as pltpu
from jax.experimental.pallas import tpu_sc as plsc

NU = 10000
NP = 10112
D = 128
CH = 128
EF, EC, ED = 96000, 160000, 32000
RPT = NP // 16
BLK = 1264
GRID = NP // BLK

_mesh = plsc.VectorSubcoreMesh(core_axis_name="c", subcore_axis_name="s")
_f32 = jnp.float32


def _chunks_for(worker, nworkers, n_chunks):
    nfull, rem = n_chunks // nworkers, n_chunks % nworkers
    return nfull + jnp.where(worker < rem, 1, 0)


def _max_chunks(nworkers, n_chunks):
    return n_chunks // nworkers + (1 if n_chunks % nworkers else 0)


_Q = 4
_QA = 3


@functools.partial(
    pl.kernel,
    out_type=[jax.ShapeDtypeStruct((NP, D), _f32)] * 6,
    mesh=_mesh,
    scratch_types=[
        pltpu.VMEM((_Q, CH), jnp.int32),
        pltpu.VMEM((CH, D), _f32),
        pltpu.VMEM_SHARED((NP, D), _f32),
    ] + [pltpu.SemaphoreType.DMA] * _Q,
)
def _deg_kernel(fs, fd, cs, cd, dsk, dd, ones_hbm, zrows_hbm,
                dfs, dfd, dcs, dcd, dds, ddd,
                idx_q, ones_v, hist, *sems):
    c = lax.axis_index("c")
    s = lax.axis_index("s")
    row0 = pl.multiple_of(s * RPT, 8)
    pltpu.sync_copy(ones_hbm, ones_v)
    plan = [
        (0, fs, EF, dfs), (0, cs, EC, dcs), (0, dsk, ED, dds),
        (1, fd, EF, dfd), (1, cd, EC, dcd), (1, dd, ED, ddd),
    ]
    for core, arr, E, out in plan:
        def run(arr=arr, E=E, out=out):
            pltpu.sync_copy(zrows_hbm, hist.at[pl.ds(row0, RPT)])
            plsc.subcore_barrier()
            n_s = _chunks_for(s, 16, E // CH)
            nq = -(-_max_chunks(16, E // CH) // _Q)

            def body(i, carry):
                conds, bases = [], []
                for k in range(_Q):
                    j = i * _Q + k
                    conds.append(j < n_s)
                    bases.append(pl.multiple_of((s + 16 * j) * CH, CH))
                for k in range(_Q):
                    def enq(k=k):
                        pltpu.async_copy(
                            arr.at[pl.ds(bases[k], CH)], idx_q.at[k], sems[k])
                    pl.when(conds[k])(enq)
                for k in range(_Q):
                    def scat(k=k):
                        pltpu.make_async_copy(
                            arr.at[pl.ds(bases[k], CH)], idx_q.at[k],
                            sems[k]).wait()
                        pltpu.async_copy(
                            ones_v, hist.at[idx_q.at[k]], sems[k], add=True)
                    pl.when(conds[k])(scat)
                for k in range(_Q):
                    def drain(k=k):
                        pltpu.make_async_copy(
                            ones_v, hist.at[idx_q.at[k]], sems[k]).wait()
                    pl.when(conds[k])(drain)
                return carry

            lax.fori_loop(0, nq, body, 0)
            plsc.subcore_barrier()
            pltpu.sync_copy(hist.at[pl.ds(row0, RPT)], out.at[pl.ds(row0, RPT)])
        pl.when(c == core)(run)


@functools.partial(
    pl.kernel,
    out_type=[jax.ShapeDtypeStruct((NP, D), _f32)] * 6,
    mesh=_mesh,
    scratch_types=[
        pltpu.VMEM((_QA, 2, CH), jnp.int32),
    ] + [pltpu.VMEM((CH, D), _f32)] * _QA + [
        pltpu.VMEM_SHARED((NP, D), _f32),
    ] + [pltpu.SemaphoreType.DMA] * _QA,
)
def _agg_kernel(t0, t1, t2, t3, t4, t5, p0, p1, p2, p3, p4, p5, zrows,
                y0, y1, y2, y3, y4, y5,
                idx_q, *rest):
    rows_l = rest[:_QA]
    acc = rest[_QA]
    sems = rest[_QA + 1:]
    c = lax.axis_index("c")
    s = lax.axis_index("s")
    row0 = pl.multiple_of(s * RPT, 8)
    plan = [
        (0, t0, p0, EF, y0),
        (0, t2, p2, EC, y2),
        (0, t5, p5, ED, y5),
        (1, t1, p1, EF, y1),
        (1, t3, p3, EC, y3),
        (1, t4, p4, ED, y4),
    ]
    for core, tbl, pk, E, yout in plan:
        def run(tbl=tbl, pk=pk, E=E, yout=yout):
            pltpu.sync_copy(zrows, acc.at[pl.ds(row0, RPT)])
            plsc.subcore_barrier()
            n_s = _chunks_for(s, 16, E // CH)
            nq = -(-_max_chunks(16, E // CH) // _QA)

            def body(i, carry):
                conds, cks = [], []
                for k in range(_QA):
                    j = i * _QA + k
                    conds.append(j < n_s)
                    cks.append(s + 16 * j)
                for k in range(_QA):
                    def enq(k=k):
                        pltpu.async_copy(pk.at[cks[k]], idx_q.at[k], sems[k])
                    pl.when(conds[k])(enq)
                for k in range(_QA):
                    def gath(k=k):
                        pltpu.make_async_copy(
                            pk.at[cks[k]], idx_q.at[k], sems[k]).wait()
                        pltpu.async_copy(
                            tbl.at[idx_q.at[k, 0]], rows_l[k], sems[k])
                    pl.when(conds[k])(gath)
                for k in range(_QA):
                    def scat(k=k):
                        pltpu.make_async_copy(
                            tbl.at[idx_q.at[k, 0]], rows_l[k], sems[k]).wait()
                        pltpu.async_copy(
                            rows_l[k], acc.at[idx_q.at[k, 1]], sems[k],
                            add=True)
                    pl.when(conds[k])(scat)
                for k in range(_QA):
                    def drain(k=k):
                        pltpu.make_async_copy(
                            rows_l[k], acc.at[idx_q.at[k, 1]], sems[k]).wait()
                    pl.when(conds[k])(drain)
                return carry

            lax.fori_loop(0, nq, body, 0)
            plsc.subcore_barrier()
            pltpu.sync_copy(acc.at[pl.ds(row0, RPT)], yout.at[pl.ds(row0, RPT)])
        pl.when(c == core)(run)


_NOUT = (EC + ED) // 8


_QP = 3


@functools.partial(
    pl.kernel,
    out_type=jax.ShapeDtypeStruct((_NOUT, D), _f32),
    mesh=_mesh,
    scratch_types=[
        pltpu.VMEM((_QP, 2, CH), jnp.int32),
    ] + [pltpu.VMEM((CH, D), _f32)] * (2 * _QP)
      + [pltpu.VMEM((CH // 8, D), _f32)] * _QP
      + [pltpu.SemaphoreType.DMA] * _QP,
)
def _pred_kernel(pu, pi, pc, pd, out,
                 idx_q, *rest):
    arows_l = rest[:_QP]
    brows_l = rest[_QP:2 * _QP]
    crows_l = rest[2 * _QP:3 * _QP]
    sems = rest[3 * _QP:]
    c = lax.axis_index("c")
    s = lax.axis_index("s")
    w = s * 2 + c
    for pk, E, obase in [(pc, EC, 0), (pd, ED, EC)]:
        def seg(pk=pk, E=E, obase=obase):
            n_w = _chunks_for(w, 32, E // CH)
            nq = -(-_max_chunks(32, E // CH) // _QP)

            def body(i, carry):
                conds, cks, bases = [], [], []
                for k in range(_QP):
                    j = i * _QP + k
                    conds.append(j < n_w)
                    cks.append(w + 32 * j)
                    bases.append(pl.multiple_of((w + 32 * j) * CH, CH))
                for k in range(_QP):
                    def enq(k=k):
                        pltpu.async_copy(pk.at[cks[k]], idx_q.at[k], sems[k])
                    pl.when(conds[k])(enq)
                for k in range(_QP):
                    def gath(k=k):
                        pltpu.make_async_copy(
                            pk.at[cks[k]], idx_q.at[k], sems[k]).wait()
                        pltpu.async_copy(
                            pu.at[idx_q.at[k, 0]], arows_l[k], sems[k])
                        pltpu.async_copy(
                            pi.at[idx_q.at[k, 1]], brows_l[k], sems[k])
                    pl.when(conds[k])(gath)
                for k in range(_QP):
                    def pack(k=k):
                        pltpu.make_async_copy(
                            pu.at[idx_q.at[k, 0]], arows_l[k], sems[k]).wait()
                        pltpu.make_async_copy(
                            pi.at[idx_q.at[k, 1]], brows_l[k], sems[k]).wait()
                        for r in range(CH):
                            crows_l[k][r // 8, pl.ds((r % 8) * 16, 16)] = (
                                arows_l[k][r, pl.ds(0, 16)]
                                + brows_l[k][r, pl.ds(0, 16)])
                        orow = pl.multiple_of((obase + bases[k]) // 8, CH // 8)
                        pltpu.async_copy(
                            crows_l[k], out.at[pl.ds(orow, CH // 8)],
                            sems[k])
                    pl.when(conds[k])(pack)
                for k in range(_QP):
                    def drain(k=k):
                        orow = pl.multiple_of((obase + bases[k]) // 8, CH // 8)
                        pltpu.make_async_copy(
                            crows_l[k], out.at[pl.ds(orow, CH // 8)],
                            sems[k]).wait()
                    pl.when(conds[k])(drain)
                return carry

            lax.fori_loop(0, nq, body, 0)
        seg()


def _tc_pre_body(xu, xi, dfs, dfd, dcs, dcd, dds, ddd,
                 t0, t1, t2, t3, t4, t5, sfs, sfd, scs, scd, sds, sdd):
    u = xu[...]
    it = xi[...]
    sv = []
    for dref, sref in [(dfs, sfs), (dfd, sfd), (dcs, scs), (dcd, scd),
                       (dds, sds), (ddd, sdd)]:
        v = lax.rsqrt(jnp.maximum(dref[...], 1.0))
        sref[...] = v
        sv.append(v)
    t0[...] = u * sv[0]
    t1[...] = u * sv[1]
    t2[...] = u * sv[2]
    t4[...] = u * sv[4]
    t3[...] = it * sv[3]
    t5[...] = it * sv[5]


def _layer_mats(y, sv, wu, wi, bu, bi):
    hu = jnp.concatenate(
        [y[0] * sv[1], y[1] * sv[0], y[3] * sv[2], y[5] * sv[4]], axis=1)
    hu = jnp.dot(hu, wu, preferred_element_type=_f32) + bu
    hi = jnp.concatenate([y[2] * sv[3], y[4] * sv[5]], axis=1)
    hi = jnp.dot(hi, wi, preferred_element_type=_f32) + bi
    return hu, hi


def _tc_mid_body(y0, y1, y2, y3, y4, y5, sfs, sfd, scs, scd, sds, sdd,
                 wu, wi, bu, bi,
                 o0, o1, o2, o3, o4, o5):
    sv = [sfs[...], sfd[...], scs[...], scd[...], sds[...], sdd[...]]
    hu, hi = _layer_mats([y0[...], y1[...], y2[...], y3[...], y4[...], y5[...]],
                         sv, wu[...], wi[...], bu[...], bi[...])
    hu = jnp.maximum(hu, 0.0)
    hi = jnp.maximum(hi, 0.0)
    o0[...] = hu * sv[0]
    o1[...] = hu * sv[1]
    o2[...] = hu * sv[2]
    o4[...] = hu * sv[4]
    o3[...] = hi * sv[3]
    o5[...] = hi * sv[5]


def _tc_post_body(y0, y1, y2, y3, y4, y5, sfs, sfd, scs, scd, sds, sdd,
                  wu, wi, bu, bi, wpu, wpi, bpp,
                  pu, pi):
    sv = [sfs[...], sfd[...], scs[...], scd[...], sds[...], sdd[...]]
    hu, hi = _layer_mats([y0[...], y1[...], y2[...], y3[...], y4[...], y5[...]],
                         sv, wu[...], wi[...], bu[...], bi[...])
    pu[...] = jnp.dot(hu, wpu[...], preferred_element_type=_f32) + bpp[...]
    pi[...] = jnp.dot(hi, wpi[...], preferred_element_type=_f32)


def _blk(shape):
    return pl.BlockSpec(shape, lambda i: (0,) * len(shape))


_rows_D = pl.BlockSpec((BLK, D), lambda i: (i, 0))
_rows_1 = pl.BlockSpec((BLK, 1), lambda i: (i, 0))
_rows_16 = pl.BlockSpec((BLK, 16), lambda i: (i, 0))

_tc_pre = pl.pallas_call(
    _tc_pre_body,
    grid=(GRID,),
    in_specs=[_rows_D] * 2 + [_rows_1] * 6,
    out_specs=[_rows_D] * 6 + [_rows_1] * 6,
    out_shape=[jax.ShapeDtypeStruct((NP, D), _f32)] * 6
              + [jax.ShapeDtypeStruct((NP, 1), _f32)] * 6,
)

_tc_mid = pl.pallas_call(
    _tc_mid_body,
    grid=(GRID,),
    in_specs=[_rows_D] * 6 + [_rows_1] * 6
             + [_blk((4 * D, D)), _blk((2 * D, D)), _blk((1, D)), _blk((1, D))],
    out_specs=[_rows_D] * 6,
    out_shape=[jax.ShapeDtypeStruct((NP, D), _f32)] * 6,
)

_tc_post = pl.pallas_call(
    _tc_post_body,
    grid=(GRID,),
    in_specs=[_rows_D] * 6 + [_rows_1] * 6
             + [_blk((4 * D, D)), _blk((2 * D, D)), _blk((1, D)), _blk((1, D)),
                _blk((D, D)), _blk((D, D)), _blk((1, D))],
    out_specs=[_rows_D] * 2,
    out_shape=[jax.ShapeDtypeStruct((NP, D), _f32)] * 2,
)


def kernel(x_user, x_item, click_src, click_dst, dislike_src, dislike_dst,
           follow_src, follow_dst, W1, b1, W2, b2, Wp, bp):
    xu = jnp.pad(x_user.astype(_f32), ((0, NP - NU), (0, 0)))
    xi = jnp.pad(x_item.astype(_f32), ((0, NP - NU), (0, 0)))
    ones128 = jnp.ones((CH, D), _f32)
    zrows = jnp.zeros((RPT, D), _f32)
    edges = (click_src, click_dst, dislike_src, dislike_dst,
             follow_src, follow_dst)
    cs, cd, dsk, dd, fs, fd = (e.astype(jnp.int32) for e in edges)

    def pack2(a, b):
        n = a.shape[0] // CH
        return jnp.stack([a.reshape(n, CH), b.reshape(n, CH)], axis=1)

    p0 = pack2(fs, fd)
    p1 = pack2(fd, fs)
    p2 = pack2(cs, cd)
    p3 = pack2(cd, cs)
    p4 = pack2(dsk, dd)
    p5 = pack2(dd, dsk)

    degs = _deg_kernel(fs, fd, cs, cd, dsk, dd, ones128, zrows)
    dcols = [dg[:, :1] for dg in degs]

    pre = _tc_pre(xu, xi, *dcols)
    tabs1, svecs = pre[:6], pre[6:]

    ys1 = _agg_kernel(*tabs1, p0, p1, p2, p3, p4, p5, zrows)

    w1u = jnp.concatenate([W1[0], W1[1], W1[3], W1[5]], axis=0)
    w1i = jnp.concatenate([W1[2], W1[4]], axis=0)
    b1u = (b1[0] + b1[1] + b1[3] + b1[5]).reshape(1, D)
    b1i = (b1[2] + b1[4]).reshape(1, D)
    tabs2 = _tc_mid(*ys1, *svecs, w1u, w1i, b1u, b1i)

    ys2 = _agg_kernel(*tabs2, p0, p1, p2, p3, p4, p5, zrows)

    w2u = jnp.concatenate([W2[0], W2[1], W2[3], W2[5]], axis=0)
    w2i = jnp.concatenate([W2[2], W2[4]], axis=0)
    b2u = (b2[0] + b2[1] + b2[3] + b2[5]).reshape(1, D)
    b2i = (b2[2] + b2[4]).reshape(1, D)
    wpu = jnp.pad(Wp[:D], ((0, 0), (0, D - 6)))
    wpi = jnp.pad(Wp[D:], ((0, 0), (0, D - 6)))
    bpp = jnp.pad(bp, (0, D - 6)).reshape(1, D)
    pu, pi = _tc_post(*ys2, *svecs, w2u, w2i, b2u, b2i, wpu, wpi, bpp)

    packed = _pred_kernel(pu, pi, p2, p4)
    return packed.reshape(EC + ED, 16)[:, :6]


# speedup vs baseline: 3.9524x; 1.0109x over previous
import functools

import jax
import jax.numpy as jnp
from jax import lax
from jax.experimental import pallas as pl
from jax.experimental.pallas import tpu as pltpu
from jax.experimental.pallas import tpu_sc as plsc

NU = 10000
NP = 10112
D = 128
CH = 128
EF, EC, ED = 96000, 160000, 32000
RPT = NP // 16
BLK = 1264
GRID = NP // BLK

_mesh = plsc.VectorSubcoreMesh(core_axis_name="c", subcore_axis_name="s")
_f32 = jnp.float32


def _chunks_for(worker, nworkers, n_chunks):
    nfull, rem = n_chunks // nworkers, n_chunks % nworkers
    return nfull + jnp.where(worker < rem, 1, 0)


def _max_chunks(nworkers, n_chunks):
    return n_chunks // nworkers + (1 if n_chunks % nworkers else 0)


_Q = 4
_QA = 3


@functools.partial(
    pl.kernel,
    out_type=[jax.ShapeDtypeStruct((NP, D), _f32)] * 6,
    mesh=_mesh,
    scratch_types=[
        pltpu.VMEM((_Q, CH), jnp.int32),
        pltpu.VMEM((CH, D), _f32),
        pltpu.VMEM_SHARED((NP, D), _f32),
    ] + [pltpu.SemaphoreType.DMA] * _Q,
)
def _deg_kernel(fs, fd, cs, cd, dsk, dd, ones_hbm, zrows_hbm,
                dfs, dfd, dcs, dcd, dds, ddd,
                idx_q, ones_v, hist, *sems):
    c = lax.axis_index("c")
    s = lax.axis_index("s")
    row0 = pl.multiple_of(s * RPT, 8)
    pltpu.sync_copy(ones_hbm, ones_v)
    plan = [
        (0, fs, EF, dfs), (0, cs, EC, dcs), (0, dsk, ED, dds),
        (1, fd, EF, dfd), (1, cd, EC, dcd), (1, dd, ED, ddd),
    ]
    for core, arr, E, out in plan:
        def run(arr=arr, E=E, out=out):
            pltpu.sync_copy(zrows_hbm, hist.at[pl.ds(row0, RPT)])
            plsc.subcore_barrier()
            n_s = _chunks_for(s, 16, E // CH)
            nq = -(-_max_chunks(16, E // CH) // _Q)

            def body(i, carry):
                conds, bases = [], []
                for k in range(_Q):
                    j = i * _Q + k
                    conds.append(j < n_s)
                    bases.append(pl.multiple_of((s + 16 * j) * CH, CH))
                for k in range(_Q):
                    def enq(k=k):
                        pltpu.async_copy(
                            arr.at[pl.ds(bases[k], CH)], idx_q.at[k], sems[k])
                    pl.when(conds[k])(enq)
                for k in range(_Q):
                    def scat(k=k):
                        pltpu.make_async_copy(
                            arr.at[pl.ds(bases[k], CH)], idx_q.at[k],
                            sems[k]).wait()
                        pltpu.async_copy(
                            ones_v, hist.at[idx_q.at[k]], sems[k], add=True)
                    pl.when(conds[k])(scat)
                for k in range(_Q):
                    def drain(k=k):
                        pltpu.make_async_copy(
                            ones_v, hist.at[idx_q.at[k]], sems[k]).wait()
                    pl.when(conds[k])(drain)
                return carry

            lax.fori_loop(0, nq, body, 0)
            plsc.subcore_barrier()
            pltpu.sync_copy(hist.at[pl.ds(row0, RPT)], out.at[pl.ds(row0, RPT)])
        pl.when(c == core)(run)


@functools.partial(
    pl.kernel,
    out_type=[jax.ShapeDtypeStruct((NP, D), _f32)] * 6,
    mesh=_mesh,
    scratch_types=[
        pltpu.VMEM((_QA, 2, CH), jnp.int32),
    ] + [pltpu.VMEM((CH, D), _f32)] * _QA + [
        pltpu.VMEM_SHARED((NP, D), _f32),
    ] + [pltpu.SemaphoreType.DMA] * _QA,
)
def _agg_kernel(t0, t1, t2, t3, t4, t5, p0, p1, p2, p3, p4, p5, zrows,
                y0, y1, y2, y3, y4, y5,
                idx_q, *rest):
    rows_l = rest[:_QA]
    acc = rest[_QA]
    sems = rest[_QA + 1:]
    c = lax.axis_index("c")
    s = lax.axis_index("s")
    row0 = pl.multiple_of(s * RPT, 8)
    plan = [
        (0, t0, p0, EF, y0),
        (0, t2, p2, EC, y2),
        (0, t5, p5, ED, y5),
        (1, t1, p1, EF, y1),
        (1, t3, p3, EC, y3),
        (1, t4, p4, ED, y4),
    ]
    for core, tbl, pk, E, yout in plan:
        def run(tbl=tbl, pk=pk, E=E, yout=yout):
            pltpu.sync_copy(zrows, acc.at[pl.ds(row0, RPT)])
            plsc.subcore_barrier()
            n_s = _chunks_for(s, 16, E // CH)
            nq = -(-_max_chunks(16, E // CH) // _QA)

            def body(i, carry):
                conds, cks = [], []
                for k in range(_QA):
                    j = i * _QA + k
                    conds.append(j < n_s)
                    cks.append(s + 16 * j)
                for k in range(_QA):
                    def enq(k=k):
                        pltpu.async_copy(pk.at[cks[k]], idx_q.at[k], sems[k])
                    pl.when(conds[k])(enq)
                for k in range(_QA):
                    def gath(k=k):
                        pltpu.make_async_copy(
                            pk.at[cks[k]], idx_q.at[k], sems[k]).wait()
                        pltpu.async_copy(
                            tbl.at[idx_q.at[k, 0]], rows_l[k], sems[k])
                    pl.when(conds[k])(gath)
                for k in range(_QA):
                    def scat(k=k):
                        pltpu.make_async_copy(
                            tbl.at[idx_q.at[k, 0]], rows_l[k], sems[k]).wait()
                        pltpu.async_copy(
                            rows_l[k], acc.at[idx_q.at[k, 1]], sems[k],
                            add=True)
                    pl.when(conds[k])(scat)
                for k in range(_QA):
                    def drain(k=k):
                        pltpu.make_async_copy(
                            rows_l[k], acc.at[idx_q.at[k, 1]], sems[k]).wait()
                    pl.when(conds[k])(drain)
                return carry

            lax.fori_loop(0, nq, body, 0)
            plsc.subcore_barrier()
            pltpu.sync_copy(acc.at[pl.ds(row0, RPT)], yout.at[pl.ds(row0, RPT)])
        pl.when(c == core)(run)


_NOUT = (EC + ED) // 8


_QP = 3


@functools.partial(
    pl.kernel,
    out_type=jax.ShapeDtypeStruct((_NOUT, D), _f32),
    mesh=_mesh,
    scratch_types=[
        pltpu.VMEM((_QP, 2, CH), jnp.int32),
    ] + [pltpu.VMEM((CH, D), _f32)] * (2 * _QP)
      + [pltpu.VMEM((CH // 8, D), _f32)] * _QP
      + [pltpu.SemaphoreType.DMA] * _QP,
)
def _pred_kernel(pu, pi, pc, pd, out,
                 idx_q, *rest):
    arows_l = rest[:_QP]
    brows_l = rest[_QP:2 * _QP]
    crows_l = rest[2 * _QP:3 * _QP]
    sems = rest[3 * _QP:]
    c = lax.axis_index("c")
    s = lax.axis_index("s")
    w = s * 2 + c
    for pk, E, obase in [(pc, EC, 0), (pd, ED, EC)]:
        def seg(pk=pk, E=E, obase=obase):
            n_w = _chunks_for(w, 32, E // CH)
            nq = -(-_max_chunks(32, E // CH) // _QP)

            def body(i, carry):
                conds, cks, bases = [], [], []
                for k in range(_QP):
                    j = i * _QP + k
                    conds.append(j < n_w)
                    cks.append(w + 32 * j)
                    bases.append(pl.multiple_of((w + 32 * j) * CH, CH))
                for k in range(_QP):
                    def enq(k=k):
                        pltpu.async_copy(pk.at[cks[k]], idx_q.at[k], sems[k])
                    pl.when(conds[k])(enq)
                for k in range(_QP):
                    def gath(k=k):
                        pltpu.make_async_copy(
                            pk.at[cks[k]], idx_q.at[k], sems[k]).wait()
                        pltpu.async_copy(
                            pu.at[idx_q.at[k, 0]], arows_l[k], sems[k])
                        pltpu.async_copy(
                            pi.at[idx_q.at[k, 1]], brows_l[k], sems[k])
                    pl.when(conds[k])(gath)
                for k in range(_QP):
                    def pack(k=k):
                        pltpu.make_async_copy(
                            pu.at[idx_q.at[k, 0]], arows_l[k], sems[k]).wait()
                        pltpu.make_async_copy(
                            pi.at[idx_q.at[k, 1]], brows_l[k], sems[k]).wait()
                        for r in range(CH):
                            crows_l[k][r // 8, pl.ds((r % 8) * 16, 16)] = (
                                arows_l[k][r, pl.ds(0, 16)]
                                + brows_l[k][r, pl.ds(0, 16)])
                        orow = pl.multiple_of((obase + bases[k]) // 8, CH // 8)
                        pltpu.async_copy(
                            crows_l[k], out.at[pl.ds(orow, CH // 8)],
                            sems[k])
                    pl.when(conds[k])(pack)
                for k in range(_QP):
                    def drain(k=k):
                        orow = pl.multiple_of((obase + bases[k]) // 8, CH // 8)
                        pltpu.make_async_copy(
                            crows_l[k], out.at[pl.ds(orow, CH // 8)],
                            sems[k]).wait()
                    pl.when(conds[k])(drain)
                return carry

            lax.fori_loop(0, nq, body, 0)
        seg()


def _tc_pre_body(xu, xi, dfs, dfd, dcs, dcd, dds, ddd,
                 t0, t1, t2, t3, t4, t5, sfs, sfd, scs, scd, sds, sdd):
    u = xu[...]
    it = xi[...]
    sv = []
    for dref, sref in [(dfs, sfs), (dfd, sfd), (dcs, scs), (dcd, scd),
                       (dds, sds), (ddd, sdd)]:
        v = lax.rsqrt(jnp.maximum(dref[...][:, :1], 1.0))
        sref[...] = v
        sv.append(v)
    t0[...] = u * sv[0]
    t1[...] = u * sv[1]
    t2[...] = u * sv[2]
    t4[...] = u * sv[4]
    t3[...] = it * sv[3]
    t5[...] = it * sv[5]


def _layer_mats(y, sv, wu, wi, bu, bi):
    hu = jnp.concatenate(
        [y[0] * sv[1], y[1] * sv[0], y[3] * sv[2], y[5] * sv[4]], axis=1)
    hu = jnp.dot(hu, wu, preferred_element_type=_f32) + bu
    hi = jnp.concatenate([y[2] * sv[3], y[4] * sv[5]], axis=1)
    hi = jnp.dot(hi, wi, preferred_element_type=_f32) + bi
    return hu, hi


def _tc_mid_body(y0, y1, y2, y3, y4, y5, sfs, sfd, scs, scd, sds, sdd,
                 wu, wi, bu, bi,
                 o0, o1, o2, o3, o4, o5):
    sv = [sfs[...], sfd[...], scs[...], scd[...], sds[...], sdd[...]]
    hu, hi = _layer_mats([y0[...], y1[...], y2[...], y3[...], y4[...], y5[...]],
                         sv, wu[...], wi[...], bu[...], bi[...])
    hu = jnp.maximum(hu, 0.0)
    hi = jnp.maximum(hi, 0.0)
    o0[...] = hu * sv[0]
    o1[...] = hu * sv[1]
    o2[...] = hu * sv[2]
    o4[...] = hu * sv[4]
    o3[...] = hi * sv[3]
    o5[...] = hi * sv[5]


def _tc_post_body(y0, y1, y2, y3, y4, y5, sfs, sfd, scs, scd, sds, sdd,
                  wu, wi, bu, bi, wpu, wpi, bpp,
                  pu, pi):
    sv = [sfs[...], sfd[...], scs[...], scd[...], sds[...], sdd[...]]
    hu, hi = _layer_mats([y0[...], y1[...], y2[...], y3[...], y4[...], y5[...]],
                         sv, wu[...], wi[...], bu[...], bi[...])
    pu[...] = jnp.dot(hu, wpu[...], preferred_element_type=_f32) + bpp[...]
    pi[...] = jnp.dot(hi, wpi[...], preferred_element_type=_f32)


def _blk(shape):
    return pl.BlockSpec(shape, lambda i: (0,) * len(shape))


_rows_D = pl.BlockSpec((BLK, D), lambda i: (i, 0))
_rows_1 = pl.BlockSpec((BLK, 1), lambda i: (i, 0))
_rows_16 = pl.BlockSpec((BLK, 16), lambda i: (i, 0))

_tc_pre = pl.pallas_call(
    _tc_pre_body,
    grid=(GRID,),
    in_specs=[_rows_D] * 8,
    out_specs=[_rows_D] * 6 + [_rows_1] * 6,
    out_shape=[jax.ShapeDtypeStruct((NP, D), _f32)] * 6
              + [jax.ShapeDtypeStruct((NP, 1), _f32)] * 6,
)

_tc_mid = pl.pallas_call(
    _tc_mid_body,
    grid=(GRID,),
    in_specs=[_rows_D] * 6 + [_rows_1] * 6
             + [_blk((4 * D, D)), _blk((2 * D, D)), _blk((1, D)), _blk((1, D))],
    out_specs=[_rows_D] * 6,
    out_shape=[jax.ShapeDtypeStruct((NP, D), _f32)] * 6,
)

_tc_post = pl.pallas_call(
    _tc_post_body,
    grid=(GRID,),
    in_specs=[_rows_D] * 6 + [_rows_1] * 6
             + [_blk((4 * D, D)), _blk((2 * D, D)), _blk((1, D)), _blk((1, D)),
                _blk((D, D)), _blk((D, D)), _blk((1, D))],
    out_specs=[_rows_D] * 2,
    out_shape=[jax.ShapeDtypeStruct((NP, D), _f32)] * 2,
)


def kernel(x_user, x_item, click_src, click_dst, dislike_src, dislike_dst,
           follow_src, follow_dst, W1, b1, W2, b2, Wp, bp):
    xu = x_user.astype(_f32)
    xi = x_item.astype(_f32)
    ones128 = jnp.ones((CH, D), _f32)
    zrows = jnp.zeros((RPT, D), _f32)
    edges = (click_src, click_dst, dislike_src, dislike_dst,
             follow_src, follow_dst)
    cs, cd, dsk, dd, fs, fd = (e.astype(jnp.int32) for e in edges)

    def pack2(a, b):
        n = a.shape[0] // CH
        return jnp.stack([a.reshape(n, CH), b.reshape(n, CH)], axis=1)

    p0 = pack2(fs, fd)
    p1 = pack2(fd, fs)
    p2 = pack2(cs, cd)
    p3 = pack2(cd, cs)
    p4 = pack2(dsk, dd)
    p5 = pack2(dd, dsk)

    degs = _deg_kernel(fs, fd, cs, cd, dsk, dd, ones128, zrows)

    pre = _tc_pre(xu, xi, *degs)
    tabs1, svecs = pre[:6], pre[6:]

    ys1 = _agg_kernel(*tabs1, p0, p1, p2, p3, p4, p5, zrows)

    w1u = jnp.concatenate([W1[0], W1[1], W1[3], W1[5]], axis=0)
    w1i = jnp.concatenate([W1[2], W1[4]], axis=0)
    b1u = (b1[0] + b1[1] + b1[3] + b1[5]).reshape(1, D)
    b1i = (b1[2] + b1[4]).reshape(1, D)
    tabs2 = _tc_mid(*ys1, *svecs, w1u, w1i, b1u, b1i)

    ys2 = _agg_kernel(*tabs2, p0, p1, p2, p3, p4, p5, zrows)

    w2u = jnp.concatenate([W2[0], W2[1], W2[3], W2[5]], axis=0)
    w2i = jnp.concatenate([W2[2], W2[4]], axis=0)
    b2u = (b2[0] + b2[1] + b2[3] + b2[5]).reshape(1, D)
    b2i = (b2[2] + b2[4]).reshape(1, D)
    wpu = jnp.pad(Wp[:D], ((0, 0), (0, D - 6)))
    wpi = jnp.pad(Wp[D:], ((0, 0), (0, D - 6)))
    bpp = jnp.pad(bp, (0, D - 6)).reshape(1, D)
    pu, pi = _tc_post(*ys2, *svecs, w2u, w2i, b2u, b2i, wpu, wpi, bpp)

    packed = _pred_kernel(pu, pi, p2, p4)
    return packed.reshape(EC + ED, 16)[:, :6]


# speedup vs baseline: 3.9747x; 1.0056x over previous
import functools

import jax
import jax.numpy as jnp
from jax import lax
from jax.experimental import pallas as pl
from jax.experimental.pallas import tpu as pltpu
from jax.experimental.pallas import tpu_sc as plsc

NU = 10000
NP = 10112
D = 128
CH = 128
EF, EC, ED = 96000, 160000, 32000
RPT = NP // 16
BLK = 1264
GRID = NP // BLK

_mesh = plsc.VectorSubcoreMesh(core_axis_name="c", subcore_axis_name="s")
_f32 = jnp.float32


def _chunks_for(worker, nworkers, n_chunks):
    nfull, rem = n_chunks // nworkers, n_chunks % nworkers
    return nfull + jnp.where(worker < rem, 1, 0)


def _max_chunks(nworkers, n_chunks):
    return n_chunks // nworkers + (1 if n_chunks % nworkers else 0)


_Q = 4
_QA = 3


@functools.partial(
    pl.kernel,
    out_type=[jax.ShapeDtypeStruct((NP, D), _f32)] * 6,
    mesh=_mesh,
    scratch_types=[
        pltpu.VMEM((_Q, CH), jnp.int32),
        pltpu.VMEM((CH, D), _f32),
        pltpu.VMEM_SHARED((NP, D), _f32),
    ] + [pltpu.SemaphoreType.DMA] * _Q,
)
def _deg_kernel(fs, fd, cs, cd, dsk, dd, ones_hbm, zrows_hbm,
                dfs, dfd, dcs, dcd, dds, ddd,
                idx_q, ones_v, hist, *sems):
    c = lax.axis_index("c")
    s = lax.axis_index("s")
    row0 = pl.multiple_of(s * RPT, 8)
    pltpu.sync_copy(ones_hbm, ones_v)
    plan = [
        (0, fs, EF, dfs), (0, cs, EC, dcs), (0, dsk, ED, dds),
        (1, fd, EF, dfd), (1, cd, EC, dcd), (1, dd, ED, ddd),
    ]
    for core, arr, E, out in plan:
        def run(arr=arr, E=E, out=out):
            pltpu.sync_copy(zrows_hbm, hist.at[pl.ds(row0, RPT)])
            plsc.subcore_barrier()
            n_s = _chunks_for(s, 16, E // CH)
            nq = -(-_max_chunks(16, E // CH) // _Q)

            def body(i, carry):
                conds, bases = [], []
                for k in range(_Q):
                    j = i * _Q + k
                    conds.append(j < n_s)
                    bases.append(pl.multiple_of((s + 16 * j) * CH, CH))
                for k in range(_Q):
                    def enq(k=k):
                        pltpu.async_copy(
                            arr.at[pl.ds(bases[k], CH)], idx_q.at[k], sems[k])
                    pl.when(conds[k])(enq)
                for k in range(_Q):
                    def scat(k=k):
                        pltpu.make_async_copy(
                            arr.at[pl.ds(bases[k], CH)], idx_q.at[k],
                            sems[k]).wait()
                        pltpu.async_copy(
                            ones_v, hist.at[idx_q.at[k]], sems[k], add=True)
                    pl.when(conds[k])(scat)
                for k in range(_Q):
                    def drain(k=k):
                        pltpu.make_async_copy(
                            ones_v, hist.at[idx_q.at[k]], sems[k]).wait()
                    pl.when(conds[k])(drain)
                return carry

            lax.fori_loop(0, nq, body, 0)
            plsc.subcore_barrier()
            pltpu.sync_copy(hist.at[pl.ds(row0, RPT)], out.at[pl.ds(row0, RPT)])
        pl.when(c == core)(run)


@functools.partial(
    pl.kernel,
    out_type=[jax.ShapeDtypeStruct((NP, D), _f32)] * 6,
    mesh=_mesh,
    scratch_types=[
        pltpu.VMEM((_QA, 2, CH), jnp.int32),
    ] + [pltpu.VMEM((CH, D), _f32)] * _QA + [
        pltpu.VMEM_SHARED((NP, D), _f32),
    ] + [pltpu.SemaphoreType.DMA] * _QA,
)
def _agg_kernel(t0, t1, t2, t3, t4, t5, p0, p1, p2, p3, p4, p5, zrows,
                y0, y1, y2, y3, y4, y5,
                idx_q, *rest):
    rows_l = rest[:_QA]
    acc = rest[_QA]
    sems = rest[_QA + 1:]
    c = lax.axis_index("c")
    s = lax.axis_index("s")
    row0 = pl.multiple_of(s * RPT, 8)
    plan = [
        (0, t0, p0, EF, y0),
        (0, t2, p2, EC, y2),
        (0, t5, p5, ED, y5),
        (1, t1, p1, EF, y1),
        (1, t3, p3, EC, y3),
        (1, t4, p4, ED, y4),
    ]
    for core, tbl, pk, E, yout in plan:
        def run(tbl=tbl, pk=pk, E=E, yout=yout):
            pltpu.sync_copy(zrows, acc.at[pl.ds(row0, RPT)])
            plsc.subcore_barrier()
            n_s = _chunks_for(s, 16, E // CH)
            nq = -(-_max_chunks(16, E // CH) // _QA)

            def body(i, carry):
                conds, cks = [], []
                for k in range(_QA):
                    j = i * _QA + k
                    conds.append(j < n_s)
                    cks.append(s + 16 * j)
                for k in range(_QA):
                    def enq(k=k):
                        pltpu.async_copy(
                            pk.at[0, cks[k]], idx_q.at[k, 0], sems[k])
                        pltpu.async_copy(
                            pk.at[1, cks[k]], idx_q.at[k, 1], sems[k])
                    pl.when(conds[k])(enq)
                for k in range(_QA):
                    def gath(k=k):
                        pltpu.make_async_copy(
                            pk.at[0, cks[k]], idx_q.at[k, 0], sems[k]).wait()
                        pltpu.make_async_copy(
                            pk.at[1, cks[k]], idx_q.at[k, 1], sems[k]).wait()
                        pltpu.async_copy(
                            tbl.at[idx_q.at[k, 0]], rows_l[k], sems[k])
                    pl.when(conds[k])(gath)
                for k in range(_QA):
                    def scat(k=k):
                        pltpu.make_async_copy(
                            tbl.at[idx_q.at[k, 0]], rows_l[k], sems[k]).wait()
                        pltpu.async_copy(
                            rows_l[k], acc.at[idx_q.at[k, 1]], sems[k],
                            add=True)
                    pl.when(conds[k])(scat)
                for k in range(_QA):
                    def drain(k=k):
                        pltpu.make_async_copy(
                            rows_l[k], acc.at[idx_q.at[k, 1]], sems[k]).wait()
                    pl.when(conds[k])(drain)
                return carry

            lax.fori_loop(0, nq, body, 0)
            plsc.subcore_barrier()
            pltpu.sync_copy(acc.at[pl.ds(row0, RPT)], yout.at[pl.ds(row0, RPT)])
        pl.when(c == core)(run)


_NOUT = (EC + ED) // 8


_QP = 3


@functools.partial(
    pl.kernel,
    out_type=jax.ShapeDtypeStruct((_NOUT, D), _f32),
    mesh=_mesh,
    scratch_types=[
        pltpu.VMEM((_QP, 2, CH), jnp.int32),
    ] + [pltpu.VMEM((CH, D), _f32)] * (2 * _QP)
      + [pltpu.VMEM((CH // 8, D), _f32)] * _QP
      + [pltpu.SemaphoreType.DMA] * _QP,
)
def _pred_kernel(pu, pi, pc, pd, out,
                 idx_q, *rest):
    arows_l = rest[:_QP]
    brows_l = rest[_QP:2 * _QP]
    crows_l = rest[2 * _QP:3 * _QP]
    sems = rest[3 * _QP:]
    c = lax.axis_index("c")
    s = lax.axis_index("s")
    w = s * 2 + c
    for pk, E, obase in [(pc, EC, 0), (pd, ED, EC)]:
        def seg(pk=pk, E=E, obase=obase):
            n_w = _chunks_for(w, 32, E // CH)
            nq = -(-_max_chunks(32, E // CH) // _QP)

            def body(i, carry):
                conds, cks, bases = [], [], []
                for k in range(_QP):
                    j = i * _QP + k
                    conds.append(j < n_w)
                    cks.append(w + 32 * j)
                    bases.append(pl.multiple_of((w + 32 * j) * CH, CH))
                for k in range(_QP):
                    def enq(k=k):
                        pltpu.async_copy(
                            pk.at[0, cks[k]], idx_q.at[k, 0], sems[k])
                        pltpu.async_copy(
                            pk.at[1, cks[k]], idx_q.at[k, 1], sems[k])
                    pl.when(conds[k])(enq)
                for k in range(_QP):
                    def gath(k=k):
                        pltpu.make_async_copy(
                            pk.at[0, cks[k]], idx_q.at[k, 0], sems[k]).wait()
                        pltpu.make_async_copy(
                            pk.at[1, cks[k]], idx_q.at[k, 1], sems[k]).wait()
                        pltpu.async_copy(
                            pu.at[idx_q.at[k, 0]], arows_l[k], sems[k])
                        pltpu.async_copy(
                            pi.at[idx_q.at[k, 1]], brows_l[k], sems[k])
                    pl.when(conds[k])(gath)
                for k in range(_QP):
                    def pack(k=k):
                        pltpu.make_async_copy(
                            pu.at[idx_q.at[k, 0]], arows_l[k], sems[k]).wait()
                        pltpu.make_async_copy(
                            pi.at[idx_q.at[k, 1]], brows_l[k], sems[k]).wait()
                        for r in range(CH):
                            crows_l[k][r // 8, pl.ds((r % 8) * 16, 16)] = (
                                arows_l[k][r, pl.ds(0, 16)]
                                + brows_l[k][r, pl.ds(0, 16)])
                        orow = pl.multiple_of((obase + bases[k]) // 8, CH // 8)
                        pltpu.async_copy(
                            crows_l[k], out.at[pl.ds(orow, CH // 8)],
                            sems[k])
                    pl.when(conds[k])(pack)
                for k in range(_QP):
                    def drain(k=k):
                        orow = pl.multiple_of((obase + bases[k]) // 8, CH // 8)
                        pltpu.make_async_copy(
                            crows_l[k], out.at[pl.ds(orow, CH // 8)],
                            sems[k]).wait()
                    pl.when(conds[k])(drain)
                return carry

            lax.fori_loop(0, nq, body, 0)
        seg()


def _tc_pre_body(xu, xi, dfs, dfd, dcs, dcd, dds, ddd,
                 t0, t1, t2, t3, t4, t5, sfs, sfd, scs, scd, sds, sdd):
    u = xu[...]
    it = xi[...]
    sv = []
    for dref, sref in [(dfs, sfs), (dfd, sfd), (dcs, scs), (dcd, scd),
                       (dds, sds), (ddd, sdd)]:
        v = lax.rsqrt(jnp.maximum(dref[...][:, :1], 1.0))
        sref[...] = v
        sv.append(v)
    t0[...] = u * sv[0]
    t1[...] = u * sv[1]
    t2[...] = u * sv[2]
    t4[...] = u * sv[4]
    t3[...] = it * sv[3]
    t5[...] = it * sv[5]


def _layer_mats(y, sv, wu, wi, bu, bi):
    hu = jnp.concatenate(
        [y[0] * sv[1], y[1] * sv[0], y[3] * sv[2], y[5] * sv[4]], axis=1)
    hu = jnp.dot(hu, wu, preferred_element_type=_f32) + bu
    hi = jnp.concatenate([y[2] * sv[3], y[4] * sv[5]], axis=1)
    hi = jnp.dot(hi, wi, preferred_element_type=_f32) + bi
    return hu, hi


def _tc_mid_body(y0, y1, y2, y3, y4, y5, sfs, sfd, scs, scd, sds, sdd,
                 wu, wi, bu, bi,
                 o0, o1, o2, o3, o4, o5):
    sv = [sfs[...], sfd[...], scs[...], scd[...], sds[...], sdd[...]]
    hu, hi = _layer_mats([y0[...], y1[...], y2[...], y3[...], y4[...], y5[...]],
                         sv, wu[...], wi[...], bu[...], bi[...])
    hu = jnp.maximum(hu, 0.0)
    hi = jnp.maximum(hi, 0.0)
    o0[...] = hu * sv[0]
    o1[...] = hu * sv[1]
    o2[...] = hu * sv[2]
    o4[...] = hu * sv[4]
    o3[...] = hi * sv[3]
    o5[...] = hi * sv[5]


def _tc_post_body(y0, y1, y2, y3, y4, y5, sfs, sfd, scs, scd, sds, sdd,
                  wu, wi, bu, bi, wpu, wpi, bpp,
                  pu, pi):
    sv = [sfs[...], sfd[...], scs[...], scd[...], sds[...], sdd[...]]
    hu, hi = _layer_mats([y0[...], y1[...], y2[...], y3[...], y4[...], y5[...]],
                         sv, wu[...], wi[...], bu[...], bi[...])
    pu[...] = jnp.dot(hu, wpu[...], preferred_element_type=_f32) + bpp[...]
    pi[...] = jnp.dot(hi, wpi[...], preferred_element_type=_f32)


def _blk(shape):
    return pl.BlockSpec(shape, lambda i: (0,) * len(shape))


_rows_D = pl.BlockSpec((BLK, D), lambda i: (i, 0))
_rows_1 = pl.BlockSpec((BLK, 1), lambda i: (i, 0))
_rows_16 = pl.BlockSpec((BLK, 16), lambda i: (i, 0))

_tc_pre = pl.pallas_call(
    _tc_pre_body,
    grid=(GRID,),
    in_specs=[_rows_D] * 8,
    out_specs=[_rows_D] * 6 + [_rows_1] * 6,
    out_shape=[jax.ShapeDtypeStruct((NP, D), _f32)] * 6
              + [jax.ShapeDtypeStruct((NP, 1), _f32)] * 6,
)

_tc_mid = pl.pallas_call(
    _tc_mid_body,
    grid=(GRID,),
    in_specs=[_rows_D] * 6 + [_rows_1] * 6
             + [_blk((4 * D, D)), _blk((2 * D, D)), _blk((1, D)), _blk((1, D))],
    out_specs=[_rows_D] * 6,
    out_shape=[jax.ShapeDtypeStruct((NP, D), _f32)] * 6,
)

_tc_post = pl.pallas_call(
    _tc_post_body,
    grid=(GRID,),
    in_specs=[_rows_D] * 6 + [_rows_1] * 6
             + [_blk((4 * D, D)), _blk((2 * D, D)), _blk((1, D)), _blk((1, D)),
                _blk((D, D)), _blk((D, D)), _blk((1, D))],
    out_specs=[_rows_D] * 2,
    out_shape=[jax.ShapeDtypeStruct((NP, D), _f32)] * 2,
)


def kernel(x_user, x_item, click_src, click_dst, dislike_src, dislike_dst,
           follow_src, follow_dst, W1, b1, W2, b2, Wp, bp):
    xu = x_user.astype(_f32)
    xi = x_item.astype(_f32)
    ones128 = jnp.ones((CH, D), _f32)
    zrows = jnp.zeros((RPT, D), _f32)
    edges = (click_src, click_dst, dislike_src, dislike_dst,
             follow_src, follow_dst)
    cs, cd, dsk, dd, fs, fd = (e.astype(jnp.int32) for e in edges)

    def pack2(a, b):
        n = a.shape[0] // CH
        return jnp.stack([a.reshape(n, CH), b.reshape(n, CH)], axis=0)

    p0 = pack2(fs, fd)
    p1 = pack2(fd, fs)
    p2 = pack2(cs, cd)
    p3 = pack2(cd, cs)
    p4 = pack2(dsk, dd)
    p5 = pack2(dd, dsk)

    degs = _deg_kernel(fs, fd, cs, cd, dsk, dd, ones128, zrows)

    pre = _tc_pre(xu, xi, *degs)
    tabs1, svecs = pre[:6], pre[6:]

    ys1 = _agg_kernel(*tabs1, p0, p1, p2, p3, p4, p5, zrows)

    w1u = jnp.concatenate([W1[0], W1[1], W1[3], W1[5]], axis=0)
    w1i = jnp.concatenate([W1[2], W1[4]], axis=0)
    b1u = (b1[0] + b1[1] + b1[3] + b1[5]).reshape(1, D)
    b1i = (b1[2] + b1[4]).reshape(1, D)
    tabs2 = _tc_mid(*ys1, *svecs, w1u, w1i, b1u, b1i)

    ys2 = _agg_kernel(*tabs2, p0, p1, p2, p3, p4, p5, zrows)

    w2u = jnp.concatenate([W2[0], W2[1], W2[3], W2[5]], axis=0)
    w2i = jnp.concatenate([W2[2], W2[4]], axis=0)
    b2u = (b2[0] + b2[1] + b2[3] + b2[5]).reshape(1, D)
    b2i = (b2[2] + b2[4]).reshape(1, D)
    wpu = jnp.pad(Wp[:D], ((0, 0), (0, D - 6)))
    wpi = jnp.pad(Wp[D:], ((0, 0), (0, D - 6)))
    bpp = jnp.pad(bp, (0, D - 6)).reshape(1, D)
    pu, pi = _tc_post(*ys2, *svecs, w2u, w2i, b2u, b2i, wpu, wpi, bpp)

    packed = _pred_kernel(pu, pi, p2, p4)
    return packed.reshape(EC + ED, 16)[:, :6]
